# baseline (speedup 1.0000x reference)
import functools

import jax
import jax.numpy as jnp
from jax import lax
from jax.experimental import pallas as pl
from jax.experimental.pallas import tpu as pltpu
from jax.experimental.pallas import tpu_sc as plsc

_TIME = 10000
_OUT = 64
_IN = 64
_ROW = _OUT * _IN


def _gather_body(mat_hbm, idx_hbm, out_hbm, idx_v, row_v, sem):
    wid = lax.axis_index("s") * 2 + lax.axis_index("c")

    @pl.when(wid == 0)
    def _():
        pltpu.sync_copy(idx_hbm, idx_v)
        pltpu.async_copy(mat_hbm.at[idx_v], row_v, sem).wait()
        pltpu.sync_copy(row_v, out_hbm)


@jax.jit
def _lookup(mat2d, idx):
    mesh = plsc.VectorSubcoreMesh(core_axis_name="c", subcore_axis_name="s")
    call = functools.partial(
        pl.kernel,
        out_type=jax.ShapeDtypeStruct((1, _ROW), jnp.float32),
        mesh=mesh,
        scratch_types=[
            pltpu.VMEM((1,), jnp.int32),
            pltpu.VMEM((1, _ROW), jnp.float32),
            pltpu.SemaphoreType.DMA,
        ],
    )(_gather_body)
    return call(mat2d, idx)


def kernel(matrix, t):
    mat2d = matrix.reshape(_TIME, _ROW)
    idx = t.astype(jnp.int32).reshape(1)
    out = _lookup(mat2d, idx)
    return out.reshape(_OUT, _IN)


# speedup vs baseline: 1.0198x; 1.0198x over previous
import functools

import jax
import jax.numpy as jnp
from jax import lax
from jax.experimental import pallas as pl
from jax.experimental.pallas import tpu as pltpu
from jax.experimental.pallas import tpu_sc as plsc

_TIME = 10000
_OUT = 64
_IN = 64
_ROW = _OUT * _IN


def _gather_body(mat_hbm, idx_hbm, out_hbm, idx_s):
    @pl.when(lax.axis_index("c") == 0)
    def _():
        pltpu.sync_copy(idx_hbm, idx_s)
        i = idx_s[0]
        pltpu.sync_copy(mat_hbm.at[i], out_hbm)


@jax.jit
def _lookup(mat2d, idx):
    mesh = plsc.ScalarSubcoreMesh(axis_name="c", num_cores=2)
    call = functools.partial(
        pl.kernel,
        out_type=jax.ShapeDtypeStruct((_ROW,), jnp.float32),
        mesh=mesh,
        scratch_types=[
            pltpu.SMEM((1,), jnp.int32),
        ],
    )(_gather_body)
    return call(mat2d, idx)


def kernel(matrix, t):
    mat2d = matrix.reshape(_TIME, _ROW)
    idx = t.astype(jnp.int32).reshape(1)
    out = _lookup(mat2d, idx)
    return out.reshape(_OUT, _IN)
